# Initial kernel scaffold; baseline (speedup 1.0000x reference)
import jax
import jax.numpy as jnp
from jax.experimental import pallas as pl


def kernel(x, edge_index, W1, b1, W2, b2):
    raise NotImplementedError("write your pallas kernel here")


# speedup vs baseline: 10.3275x; 10.3275x over previous
import functools

import jax
import jax.numpy as jnp
from jax import lax
from jax.experimental import pallas as pl
from jax.experimental.pallas import tpu as pltpu, tpu_sc as plsc

NC = 2
NS = 16
NW = NC * NS
CH = 128
DW = 16


def _sc_mesh():
  return plsc.VectorSubcoreMesh(
      core_axis_name="c", subcore_axis_name="s", num_cores=NC,
      num_subcores=NS)


def _make_deg_kernel(EP, NP, cpw):
  rpt = NP // NS

  @functools.partial(
      pl.kernel,
      mesh=_sc_mesh(),
      out_type=jax.ShapeDtypeStruct((NC * NP, DW), jnp.float32),
      scratch_types=[
          pltpu.VMEM((CH,), jnp.int32),
          pltpu.VMEM((CH, DW), jnp.float32),
          pltpu.VMEM_SHARED((NP, DW), jnp.float32),
      ],
  )
  def deg_kernel(dst_hbm, ones_hbm, zeros_hbm, degp_hbm, idx_v, ones_v,
                 acc_sh):
    c = lax.axis_index("c")
    s = lax.axis_index("s")
    wid = s * NC + c
    r0 = pl.multiple_of(s * rpt, 8)
    pltpu.sync_copy(zeros_hbm.at[pl.ds(r0, rpt)], acc_sh.at[pl.ds(r0, rpt)])
    pltpu.sync_copy(ones_hbm, ones_v)
    plsc.subcore_barrier()
    base = wid * (cpw * CH)

    def body(j, carry):
      st = pl.multiple_of(base + j * CH, CH)
      pltpu.sync_copy(dst_hbm.at[pl.ds(st, CH)], idx_v)
      pltpu.sync_copy(ones_v, acc_sh.at[idx_v], add=True)
      return carry

    lax.fori_loop(0, cpw, body, 0)
    plsc.subcore_barrier()
    o0 = pl.multiple_of(c * NP + s * rpt, 8)
    pltpu.sync_copy(acc_sh.at[pl.ds(r0, rpt)], degp_hbm.at[pl.ds(o0, rpt)])

  return deg_kernel


def _make_scatter_kernel(EP, NP, N, D, cpw):
  rpt = NP // NS

  @functools.partial(
      pl.kernel,
      mesh=_sc_mesh(),
      out_type=jax.ShapeDtypeStruct((NC * NP, D), jnp.float32),
      scratch_types=[
          pltpu.VMEM((CH,), jnp.int32),
          pltpu.VMEM((CH,), jnp.int32),
          pltpu.VMEM((CH, D), jnp.float32),
          pltpu.VMEM_SHARED((NP, D), jnp.float32),
          pltpu.SemaphoreType.DMA,
      ],
  )
  def scat_kernel(src_hbm, dst_hbm, g_hbm, zeros_hbm, part_hbm, sidx_v,
                  didx_v, rows_v, acc_sh, sem):
    c = lax.axis_index("c")
    s = lax.axis_index("s")
    wid = s * NC + c
    r0 = pl.multiple_of(s * rpt, 8)
    pltpu.sync_copy(zeros_hbm.at[pl.ds(r0, rpt)], acc_sh.at[pl.ds(r0, rpt)])
    plsc.subcore_barrier()
    base = wid * (cpw * CH)

    def body(j, carry):
      st = pl.multiple_of(base + j * CH, CH)
      pltpu.sync_copy(src_hbm.at[pl.ds(st, CH)], sidx_v)
      pltpu.sync_copy(dst_hbm.at[pl.ds(st, CH)], didx_v)
      pltpu.async_copy(g_hbm.at[sidx_v], rows_v, sem).wait()
      pltpu.sync_copy(rows_v, acc_sh.at[didx_v], add=True)
      return carry

    lax.fori_loop(0, cpw, body, 0)
    plsc.subcore_barrier()
    o0 = pl.multiple_of(c * NP + s * rpt, 8)
    pltpu.sync_copy(acc_sh.at[pl.ds(r0, rpt)], part_hbm.at[pl.ds(o0, rpt)])

  return scat_kernel


def _mm_body(x_ref, w_ref, o_ref):
  o_ref[:] = jnp.dot(x_ref[:], w_ref[:], preferred_element_type=jnp.float32)


def _dis_body(dega_ref, degb_ref, p1_ref, dis_ref, g1_ref):
  d = dega_ref[:] + degb_ref[:] + 1.0
  dv = lax.rsqrt(d)
  dis_ref[:] = dv
  g1_ref[:] = p1_ref[:] * dv


def _mid_body(pa_ref, pb_ref, g1_ref, dis_ref, b1_ref, w2_ref, g2_ref):
  dv = dis_ref[:]
  h = (pa_ref[:] + pb_ref[:] + g1_ref[:]) * dv + b1_ref[:]
  h = jnp.maximum(h, 0.0)
  g2_ref[:] = jnp.dot(h, w2_ref[:], preferred_element_type=jnp.float32) * dv


def _fin_body(pa_ref, pb_ref, g2_ref, dis_ref, b2_ref, o_ref):
  o_ref[:] = (pa_ref[:] + pb_ref[:] + g2_ref[:]) * dis_ref[:] + b2_ref[:]


def kernel(x, edge_index, W1, b1, W2, b2):
  N, D = x.shape
  E = edge_index.shape[1]

  ei = edge_index.astype(jnp.int32)
  src, dst = ei[0], ei[1]

  NP = ((N + NS - 1) // NS) * NS
  if (NP // NS) % 8 != 0:
    NP = ((N + 8 * NS - 1) // (8 * NS)) * (8 * NS)
  cpw = (E + NW * CH - 1) // (NW * CH)
  EP = cpw * NW * CH
  if EP > E:
    pad = EP - E
    src = jnp.concatenate([src, jnp.zeros((pad,), jnp.int32)])
    dst = jnp.concatenate([dst, jnp.full((pad,), N, jnp.int32)])

  zeros_d = jnp.zeros((NP, D), jnp.float32)
  zeros_w = jnp.zeros((NP, DW), jnp.float32)
  ones_w = jnp.ones((CH, DW), jnp.float32)

  deg_k = _make_deg_kernel(EP, NP, cpw)
  scat_k = _make_scatter_kernel(EP, NP, N, D, cpw)

  BR = 1000 if N % 1000 == 0 else 8
  grid = (pl.cdiv(N, BR),)
  row_spec = pl.BlockSpec((BR, D), lambda i: (i, 0))
  col_spec = pl.BlockSpec((BR, 1), lambda i: (i, 0))
  full_spec = pl.BlockSpec((D, D), lambda i: (0, 0))
  bias_spec = pl.BlockSpec((1, D), lambda i: (0, 0))

  p1 = pl.pallas_call(
      _mm_body,
      grid=grid,
      in_specs=[row_spec, full_spec],
      out_specs=row_spec,
      out_shape=jax.ShapeDtypeStruct((N, D), jnp.float32),
  )(x, W1)

  degp = deg_k(dst, ones_w, zeros_w)
  dega = degp[:N, :1]
  degb = degp[NP:NP + N, :1]

  dis, g1 = pl.pallas_call(
      _dis_body,
      grid=grid,
      in_specs=[col_spec, col_spec, row_spec],
      out_specs=[col_spec, row_spec],
      out_shape=[
          jax.ShapeDtypeStruct((N, 1), jnp.float32),
          jax.ShapeDtypeStruct((N, D), jnp.float32),
      ],
  )(dega, degb, p1)

  part1 = scat_k(src, dst, g1, zeros_d)

  g2 = pl.pallas_call(
      _mid_body,
      grid=grid,
      in_specs=[row_spec, row_spec, row_spec, col_spec, bias_spec, full_spec],
      out_specs=row_spec,
      out_shape=jax.ShapeDtypeStruct((N, D), jnp.float32),
  )(part1[:N], part1[NP:NP + N], g1, dis, b1.reshape(1, D), W2)

  part2 = scat_k(src, dst, g2, zeros_d)

  out = pl.pallas_call(
      _fin_body,
      grid=grid,
      in_specs=[row_spec, row_spec, row_spec, col_spec, bias_spec],
      out_specs=row_spec,
      out_shape=jax.ShapeDtypeStruct((N, D), jnp.float32),
  )(part2[:N], part2[NP:NP + N], g2, dis, b2.reshape(1, D))

  return out
